# baseline (device time: 728050 ns/iter reference)
import jax
import jax.numpy as jnp
from jax import lax
from jax.experimental import pallas as pl
from jax.experimental.pallas import tpu as pltpu

N_DEV = 8
CH = 512
HH = CH // 2
SEG = 2
SH = HH // SEG


def kernel(x, w_mat):
    m, _ = x.shape
    _, n = w_mat.shape

    x = x.astype(jnp.bfloat16)
    w_mat = w_mat.astype(jnp.bfloat16)

    def body(x_ref, w_ref, dummy_ref, out_ref,
             rbuf_r, sbuf_r, ssem_r, rsem_r, credit_r, osem_r,
             rbuf_l, sbuf_l, ssem_l, rsem_l, credit_l, osem_l):
        i = lax.axis_index("i")
        left = lax.rem(i - 1 + N_DEV, N_DEV)
        right = lax.rem(i + 1, N_DEV)

        barrier = pltpu.get_barrier_semaphore()
        for nbr in (left, right):
            pl.semaphore_signal(barrier, 1, device_id=(nbr,),
                                device_id_type=pl.DeviceIdType.MESH)
        pl.semaphore_wait(barrier, 2)

        class Ring:

            def __init__(self, d, rbuf, sbuf, ssem, rsem, credit, osem,
                         dst, sender, row_off):
                self.d = d
                self.rbuf, self.sbuf = rbuf, sbuf
                self.ssem, self.rsem = ssem, rsem
                self.credit, self.osem = credit, osem
                self.dst, self.sender = dst, sender
                self.row_off = row_off
                self.inflight = [[None] * SEG, [None] * SEG]
                self.swaited = [[True] * SEG, [True] * SEG]
                self.pstore = [[None] * SEG, [None] * SEG]

            def seed(self):
                pl.semaphore_signal(self.credit, 2 * SEG,
                                    device_id=(self.sender,),
                                    device_id_type=pl.DeviceIdType.MESH)

            def give_credit(self):
                pl.semaphore_signal(self.credit, 1, device_id=(self.sender,),
                                    device_id_type=pl.DeviceIdType.MESH)

            def partial(self, c):
                rows = pl.ds(c * CH + self.row_off, HH)
                return jnp.dot(x_ref[rows, :], w_ref[:, :],
                               preferred_element_type=jnp.float32
                               ).astype(jnp.bfloat16)

            def send_seg(self, step, seg):
                slot = step % 2
                assert self.swaited[slot][seg]
                pl.semaphore_wait(self.credit, 1)
                rdma = pltpu.make_async_remote_copy(
                    src_ref=self.sbuf.at[slot, seg],
                    dst_ref=self.rbuf.at[slot, seg],
                    send_sem=self.ssem.at[slot, seg],
                    recv_sem=self.rsem.at[slot, seg],
                    device_id=(self.dst,),
                    device_id_type=pl.DeviceIdType.MESH)
                rdma.start()
                self.inflight[slot][seg] = rdma
                self.swaited[slot][seg] = False

            def wait_recv(self, step, seg):
                self.inflight[step % 2][seg].wait_recv()

            def wait_sbuf_free(self, slot, seg):
                if (self.inflight[slot][seg] is not None
                        and not self.swaited[slot][seg]):
                    self.inflight[slot][seg].wait_send()
                    self.swaited[slot][seg] = True
                if self.pstore[slot][seg] is not None:
                    self.pstore[slot][seg].wait()
                    self.pstore[slot][seg] = None

            def store_out(self, slot, seg, c):
                cp = pltpu.make_async_copy(
                    self.sbuf.at[slot, seg],
                    out_ref.at[pl.ds(c * CH + self.row_off + seg * SH, SH), :],
                    self.osem.at[slot, seg])
                cp.start()
                self.pstore[slot][seg] = cp

            def drain(self):
                pl.semaphore_wait(self.credit, 2 * SEG)
                for slot in (0, 1):
                    for seg in range(SEG):
                        self.wait_sbuf_free(slot, seg)

            def rs_recv_chunk(self, s):
                return lax.rem(i - self.d * (1 + s) + 2 * N_DEV, N_DEV)

            def own_chunk(self):
                return lax.rem(i + self.d + N_DEV, N_DEV)

            def ag_recv_chunk(self, h):
                return lax.rem(i - self.d * h + 2 * N_DEV, N_DEV)

        rings = [
            Ring(+1, rbuf_r, sbuf_r, ssem_r, rsem_r, credit_r, osem_r,
                 dst=right, sender=left, row_off=0),
            Ring(-1, rbuf_l, sbuf_l, ssem_l, rsem_l, credit_l, osem_l,
                 dst=left, sender=right, row_off=HH),
        ]

        for r in rings:
            r.seed()

        gc = 0.7978845608028654

        def gelu(y):
            return 0.5 * y * (1.0 + jnp.tanh(gc * (y + 0.044715 * y * y * y)))

        def seg_rows(seg):
            return slice(seg * SH, (seg + 1) * SH)

        for r in rings:
            p = r.partial(i)
            for seg in range(SEG):
                r.sbuf[0, seg, :, :] = p[seg_rows(seg), :]
        for seg in range(SEG):
            for r in rings:
                r.send_seg(0, seg)
        parts = [r.partial(r.rs_recv_chunk(0)) for r in rings]

        for s in range(N_DEV - 1):
            slot, ns = s % 2, (s + 1) % 2
            last = s == N_DEV - 2
            for seg in range(SEG):
                for k, r in enumerate(rings):
                    r.wait_recv(s, seg)
                    acc = parts[k][seg_rows(seg), :] + r.rbuf[slot, seg, :, :]
                    r.wait_sbuf_free(ns, seg)
                    if last:
                        y = gelu(acc.astype(jnp.float32))
                        r.sbuf[ns, seg, :, :] = y.astype(jnp.bfloat16)
                        r.give_credit()
                        r.store_out(ns, seg, r.own_chunk())
                    else:
                        r.sbuf[ns, seg, :, :] = acc
                        r.give_credit()
                for r in rings:
                    r.send_seg(s + 1, seg)
            if not last:
                parts = [r.partial(r.rs_recv_chunk(s + 1)) for r in rings]

        for h in range(N_DEV - 1):
            t = N_DEV - 1 + h
            slot, ns = t % 2, (t + 1) % 2
            for seg in range(SEG):
                for k, r in enumerate(rings):
                    r.wait_recv(t, seg)
                    r.wait_sbuf_free(ns, seg)
                    r.sbuf[ns, seg, :, :] = r.rbuf[slot, seg, :, :]
                    r.give_credit()
                    r.store_out(ns, seg, r.ag_recv_chunk(h))
                if h < N_DEV - 2:
                    for r in rings:
                        r.send_seg(t + 1, seg)

        for r in rings:
            r.drain()

    ring_scratch = [
        pltpu.VMEM((2, SEG, SH, n), jnp.bfloat16),
        pltpu.VMEM((2, SEG, SH, n), jnp.bfloat16),
        pltpu.SemaphoreType.DMA((2, SEG)),
        pltpu.SemaphoreType.DMA((2, SEG)),
        pltpu.SemaphoreType.REGULAR,
        pltpu.SemaphoreType.DMA((2, SEG)),
    ]
    dummy = jnp.zeros((m, n), jnp.bfloat16)
    return pl.pallas_call(
        body,
        out_shape=jax.ShapeDtypeStruct((m, n), jnp.bfloat16),
        in_specs=[pl.BlockSpec(memory_space=pltpu.VMEM),
                  pl.BlockSpec(memory_space=pltpu.VMEM),
                  pl.BlockSpec(memory_space=pl.ANY)],
        out_specs=pl.BlockSpec(memory_space=pl.ANY),
        scratch_shapes=ring_scratch + ring_scratch,
        input_output_aliases={2: 0},
        compiler_params=pltpu.CompilerParams(
            collective_id=0, vmem_limit_bytes=100 * 1024 * 1024),
    )(x, w_mat, dummy)


# device time: 705715 ns/iter; 1.0316x vs baseline; 1.0316x over previous
import jax
import jax.numpy as jnp
from jax import lax
from jax.experimental import pallas as pl
from jax.experimental.pallas import tpu as pltpu

N_DEV = 8
CH = 512
HH = CH // 2
SEG = 2
SH = HH // SEG


def kernel(x, w_mat):
    m, _ = x.shape
    _, n = w_mat.shape

    x = x.astype(jnp.bfloat16)
    w_mat = w_mat.astype(jnp.bfloat16)

    def body(x_ref, w_ref, out_ref,
             rbuf_r, sbuf_r, ssem_r, rsem_r, credit_r, osem_r,
             rbuf_l, sbuf_l, ssem_l, rsem_l, credit_l, osem_l):
        i = lax.axis_index("i")
        left = lax.rem(i - 1 + N_DEV, N_DEV)
        right = lax.rem(i + 1, N_DEV)

        barrier = pltpu.get_barrier_semaphore()
        for nbr in (left, right):
            pl.semaphore_signal(barrier, 1, device_id=(nbr,),
                                device_id_type=pl.DeviceIdType.MESH)
        pl.semaphore_wait(barrier, 2)

        class Ring:

            def __init__(self, d, rbuf, sbuf, ssem, rsem, credit, osem,
                         dst, sender, row_off):
                self.d = d
                self.rbuf, self.sbuf = rbuf, sbuf
                self.ssem, self.rsem = ssem, rsem
                self.credit, self.osem = credit, osem
                self.dst, self.sender = dst, sender
                self.row_off = row_off
                self.inflight = [[None] * SEG, [None] * SEG]
                self.swaited = [[True] * SEG, [True] * SEG]
                self.pstore = [[None] * SEG, [None] * SEG]

            def seed(self):
                pl.semaphore_signal(self.credit, 2 * SEG,
                                    device_id=(self.sender,),
                                    device_id_type=pl.DeviceIdType.MESH)

            def give_credit(self):
                pl.semaphore_signal(self.credit, 1, device_id=(self.sender,),
                                    device_id_type=pl.DeviceIdType.MESH)

            def partial(self, c):
                rows = pl.ds(c * CH + self.row_off, HH)
                return jnp.dot(x_ref[rows, :], w_ref[:, :],
                               preferred_element_type=jnp.float32
                               ).astype(jnp.bfloat16)

            def send_seg(self, step, seg):
                slot = step % 2
                assert self.swaited[slot][seg]
                pl.semaphore_wait(self.credit, 1)
                rdma = pltpu.make_async_remote_copy(
                    src_ref=self.sbuf.at[slot, seg],
                    dst_ref=self.rbuf.at[slot, seg],
                    send_sem=self.ssem.at[slot, seg],
                    recv_sem=self.rsem.at[slot, seg],
                    device_id=(self.dst,),
                    device_id_type=pl.DeviceIdType.MESH)
                rdma.start()
                self.inflight[slot][seg] = rdma
                self.swaited[slot][seg] = False

            def wait_recv(self, step, seg):
                self.inflight[step % 2][seg].wait_recv()

            def wait_sbuf_free(self, slot, seg):
                if (self.inflight[slot][seg] is not None
                        and not self.swaited[slot][seg]):
                    self.inflight[slot][seg].wait_send()
                    self.swaited[slot][seg] = True
                if self.pstore[slot][seg] is not None:
                    self.pstore[slot][seg].wait()
                    self.pstore[slot][seg] = None

            def store_out(self, slot, seg, c):
                cp = pltpu.make_async_copy(
                    self.sbuf.at[slot, seg],
                    out_ref.at[pl.ds(c * CH + self.row_off + seg * SH, SH), :],
                    self.osem.at[slot, seg])
                cp.start()
                self.pstore[slot][seg] = cp

            def drain(self):
                pl.semaphore_wait(self.credit, 2 * SEG)
                for slot in (0, 1):
                    for seg in range(SEG):
                        self.wait_sbuf_free(slot, seg)

            def rs_recv_chunk(self, s):
                return lax.rem(i - self.d * (1 + s) + 2 * N_DEV, N_DEV)

            def own_chunk(self):
                return lax.rem(i + self.d + N_DEV, N_DEV)

            def ag_recv_chunk(self, h):
                return lax.rem(i - self.d * h + 2 * N_DEV, N_DEV)

        rings = [
            Ring(+1, rbuf_r, sbuf_r, ssem_r, rsem_r, credit_r, osem_r,
                 dst=right, sender=left, row_off=0),
            Ring(-1, rbuf_l, sbuf_l, ssem_l, rsem_l, credit_l, osem_l,
                 dst=left, sender=right, row_off=HH),
        ]

        for r in rings:
            r.seed()

        gc = 0.7978845608028654

        def gelu(y):
            return 0.5 * y * (1.0 + jnp.tanh(gc * (y + 0.044715 * y * y * y)))

        def seg_rows(seg):
            return slice(seg * SH, (seg + 1) * SH)

        for r in rings:
            p = r.partial(i)
            for seg in range(SEG):
                r.sbuf[0, seg, :, :] = p[seg_rows(seg), :]
        for seg in range(SEG):
            for r in rings:
                r.send_seg(0, seg)
        parts = [r.partial(r.rs_recv_chunk(0)) for r in rings]

        for s in range(N_DEV - 1):
            slot, ns = s % 2, (s + 1) % 2
            last = s == N_DEV - 2
            for seg in range(SEG):
                for k, r in enumerate(rings):
                    r.wait_recv(s, seg)
                    acc = parts[k][seg_rows(seg), :] + r.rbuf[slot, seg, :, :]
                    r.wait_sbuf_free(ns, seg)
                    if last:
                        y = gelu(acc.astype(jnp.float32))
                        r.sbuf[ns, seg, :, :] = y.astype(jnp.bfloat16)
                        r.give_credit()
                        r.store_out(ns, seg, r.own_chunk())
                    else:
                        r.sbuf[ns, seg, :, :] = acc
                        r.give_credit()
                for r in rings:
                    r.send_seg(s + 1, seg)
            if not last:
                parts = [r.partial(r.rs_recv_chunk(s + 1)) for r in rings]

        for h in range(N_DEV - 1):
            t = N_DEV - 1 + h
            slot, ns = t % 2, (t + 1) % 2
            for seg in range(SEG):
                for k, r in enumerate(rings):
                    r.wait_recv(t, seg)
                    r.wait_sbuf_free(ns, seg)
                    r.sbuf[ns, seg, :, :] = r.rbuf[slot, seg, :, :]
                    r.give_credit()
                    r.store_out(ns, seg, r.ag_recv_chunk(h))
                if h < N_DEV - 2:
                    for r in rings:
                        r.send_seg(t + 1, seg)

        for r in rings:
            r.drain()

    ring_scratch = [
        pltpu.VMEM((2, SEG, SH, n), jnp.bfloat16),
        pltpu.VMEM((2, SEG, SH, n), jnp.bfloat16),
        pltpu.SemaphoreType.DMA((2, SEG)),
        pltpu.SemaphoreType.DMA((2, SEG)),
        pltpu.SemaphoreType.REGULAR,
        pltpu.SemaphoreType.DMA((2, SEG)),
    ]
    return pl.pallas_call(
        body,
        out_shape=jax.ShapeDtypeStruct((m, n), jnp.bfloat16),
        in_specs=[pl.BlockSpec(memory_space=pltpu.VMEM),
                  pl.BlockSpec(memory_space=pltpu.VMEM)],
        out_specs=pl.BlockSpec(memory_space=pl.ANY),
        scratch_shapes=ring_scratch + ring_scratch,
        compiler_params=pltpu.CompilerParams(
            collective_id=0, vmem_limit_bytes=100 * 1024 * 1024),
    )(x, w_mat)
